# baseline (speedup 1.0000x reference)
import jax
import jax.numpy as jnp
from jax.experimental import pallas as pl
from jax.experimental.pallas import tpu as pltpu

_BLOCK_ROWS = 2000


def _fused_cell(x_ref, wz_ref, bz_ref, wh_ref, bh_ref, wlr_ref, wlc_ref,
                bl_ref, o_ref):
    xb = x_ref[...].astype(jnp.bfloat16)
    f = xb.shape[1]
    hd = wz_ref.shape[-1]
    w0 = jnp.concatenate([wz_ref[0, 0, :f, :], wh_ref[0, 0, :f, :]],
                         axis=1).astype(jnp.bfloat16)
    w1 = jnp.concatenate([wz_ref[1, 0, :f, :], wh_ref[1, 0, :f, :]],
                         axis=1).astype(jnp.bfloat16)
    bcat = jnp.concatenate([bz_ref[...], bh_ref[...]], axis=1)
    g = (jnp.dot(xb, w0, preferred_element_type=jnp.float32)
         + jnp.dot(xb, w1, preferred_element_type=jnp.float32) + bcat)
    z = jax.nn.sigmoid(g[:, :hd])
    ht = jnp.tanh(g[:, hd:])
    h = (1.0 - z) * ht
    r = jnp.maximum(h, 0.0)
    c = jax.nn.sigmoid(h)
    r16 = r.astype(jnp.bfloat16).astype(jnp.float32)
    c16 = c.astype(jnp.bfloat16).astype(jnp.float32)
    wlr = wlr_ref[...].astype(jnp.bfloat16).astype(jnp.float32)
    wlc = wlc_ref[...].astype(jnp.bfloat16).astype(jnp.float32)
    acc = r16 * wlr + c16 * wlc
    out = jnp.sum(acc, axis=1, keepdims=True) + bl_ref[...]
    o_ref[...] = jnp.maximum(out, 0.0)


def kernel(x, edge_index, edge_weight, W_z, b_z, W_r, b_r, W_h, b_h,
           W_lin, b_lin):
    del edge_index, edge_weight, W_r, b_r
    n, f = x.shape
    hdim = W_z.shape[-1]
    block = _BLOCK_ROWS if n % _BLOCK_ROWS == 0 else n
    grid = n // block

    bz2 = b_z.reshape(1, hdim)
    bh2 = b_h.reshape(1, hdim)
    wl_r = W_lin[:hdim, 0].reshape(1, hdim)
    wl_c = W_lin[hdim:, 0].reshape(1, hdim)
    bl2 = b_lin.reshape(1, 1)

    full = lambda a: pl.BlockSpec(a.shape, lambda i: (0,) * a.ndim)
    return pl.pallas_call(
        _fused_cell,
        grid=(grid,),
        in_specs=[
            pl.BlockSpec((block, f), lambda i: (i, 0)),
            full(W_z), full(bz2), full(W_h), full(bh2),
            full(wl_r), full(wl_c), full(bl2),
        ],
        out_specs=pl.BlockSpec((block, 1), lambda i: (i, 0)),
        out_shape=jax.ShapeDtypeStruct((n, 1), jnp.float32),
        compiler_params=pltpu.CompilerParams(
            dimension_semantics=("parallel",)),
    )(x, W_z, bz2, W_h, bh2, wl_r, wl_c, bl2)


# speedup vs baseline: 1.2983x; 1.2983x over previous
import jax
import jax.numpy as jnp
from jax.experimental import pallas as pl
from jax.experimental.pallas import tpu as pltpu

_BLOCK_ROWS = 1000

_DEFAULT = jax.lax.Precision.DEFAULT


def _fused_cell(x_ref, wz_ref, bz_ref, wh_ref, bh_ref, wl_ref, bl_ref,
                o_ref):
    xb = x_ref[...]
    f = xb.shape[1]
    hd = wz_ref.shape[-1]
    w0 = jnp.concatenate([wz_ref[0, 0, :f, :], wh_ref[0, 0, :f, :]], axis=1)
    w1 = jnp.concatenate([wz_ref[1, 0, :f, :], wh_ref[1, 0, :f, :]], axis=1)
    bcat = jnp.concatenate([bz_ref[...], bh_ref[...]], axis=1)
    g = (jnp.dot(xb, w0, precision=_DEFAULT,
                 preferred_element_type=jnp.float32)
         + jnp.dot(xb, w1, precision=_DEFAULT,
                   preferred_element_type=jnp.float32) + bcat)
    z = jax.nn.sigmoid(g[:, :hd])
    ht = jnp.tanh(g[:, hd:])
    h = (1.0 - z) * ht
    hcat = jnp.concatenate([jnp.maximum(h, 0.0), jax.nn.sigmoid(h)], axis=1)
    out = jnp.dot(hcat, wl_ref[...], precision=_DEFAULT,
                  preferred_element_type=jnp.float32) + bl_ref[...]
    o_ref[...] = jnp.maximum(out, 0.0)


def kernel(x, edge_index, edge_weight, W_z, b_z, W_r, b_r, W_h, b_h,
           W_lin, b_lin):
    del edge_index, edge_weight, W_r, b_r
    n, f = x.shape
    hdim = W_z.shape[-1]
    block = _BLOCK_ROWS if n % _BLOCK_ROWS == 0 else n
    grid = n // block

    bz2 = b_z.reshape(1, hdim)
    bh2 = b_h.reshape(1, hdim)
    bl2 = b_lin.reshape(1, 1)

    full = lambda a: pl.BlockSpec(a.shape, lambda i: (0,) * a.ndim)
    return pl.pallas_call(
        _fused_cell,
        grid=(grid,),
        in_specs=[
            pl.BlockSpec((block, f), lambda i: (i, 0)),
            full(W_z), full(bz2), full(W_h), full(bh2),
            full(W_lin), full(bl2),
        ],
        out_specs=pl.BlockSpec((block, 1), lambda i: (i, 0)),
        out_shape=jax.ShapeDtypeStruct((n, 1), jnp.float32),
        compiler_params=pltpu.CompilerParams(
            dimension_semantics=("parallel",)),
    )(x, W_z, bz2, W_h, bh2, W_lin, bl2)


# speedup vs baseline: 1.5333x; 1.1810x over previous
import jax
import jax.numpy as jnp
from jax.experimental import pallas as pl
from jax.experimental.pallas import tpu as pltpu

_BLOCK_ROWS = 2000

_DEFAULT = jax.lax.Precision.DEFAULT


def _fused_cell(x_ref, wz_ref, bz_ref, wh_ref, bh_ref, wl_ref, bl_ref,
                o_ref):
    xb = x_ref[...]
    f = xb.shape[1]
    hd = wz_ref.shape[-1]
    w0 = jnp.concatenate([wz_ref[0, 0, :f, :], wh_ref[0, 0, :f, :]], axis=1)
    w1 = jnp.concatenate([wz_ref[1, 0, :f, :], wh_ref[1, 0, :f, :]], axis=1)
    bcat = jnp.concatenate([bz_ref[...], bh_ref[...]], axis=1)
    g = (jnp.dot(xb, w0, precision=_DEFAULT,
                 preferred_element_type=jnp.float32)
         + jnp.dot(xb, w1, precision=_DEFAULT,
                   preferred_element_type=jnp.float32) + bcat)
    z = jax.nn.sigmoid(g[:, :hd])
    ht = jnp.tanh(g[:, hd:])
    h = (1.0 - z) * ht
    hcat = jnp.concatenate([jnp.maximum(h, 0.0), jax.nn.sigmoid(h)], axis=1)
    out = jnp.dot(hcat, wl_ref[...], precision=_DEFAULT,
                  preferred_element_type=jnp.float32) + bl_ref[...]
    o_ref[...] = jnp.maximum(out, 0.0)


def kernel(x, edge_index, edge_weight, W_z, b_z, W_r, b_r, W_h, b_h,
           W_lin, b_lin):
    del edge_index, edge_weight, W_r, b_r
    n, f = x.shape
    hdim = W_z.shape[-1]
    block = _BLOCK_ROWS if n % _BLOCK_ROWS == 0 else n
    grid = n // block

    bz2 = b_z.reshape(1, hdim)
    bh2 = b_h.reshape(1, hdim)
    bl2 = b_lin.reshape(1, 1)

    full = lambda a: pl.BlockSpec(a.shape, lambda i: (0,) * a.ndim)
    return pl.pallas_call(
        _fused_cell,
        grid=(grid,),
        in_specs=[
            pl.BlockSpec((block, f), lambda i: (i, 0)),
            full(W_z), full(bz2), full(W_h), full(bh2),
            full(W_lin), full(bl2),
        ],
        out_specs=pl.BlockSpec((block, 1), lambda i: (i, 0)),
        out_shape=jax.ShapeDtypeStruct((n, 1), jnp.float32),
        compiler_params=pltpu.CompilerParams(
            dimension_semantics=("parallel",)),
    )(x, W_z, bz2, W_h, bh2, W_lin, bl2)


# speedup vs baseline: 1.5368x; 1.0022x over previous
import jax
import jax.numpy as jnp
from jax.experimental import pallas as pl
from jax.experimental.pallas import tpu as pltpu

_BLOCK_ROWS = 5000

_DEFAULT = jax.lax.Precision.DEFAULT


def _fused_cell(x_ref, wz_ref, bz_ref, wh_ref, bh_ref, wl_ref, bl_ref,
                o_ref):
    xb = x_ref[...]
    f = xb.shape[1]
    hd = wz_ref.shape[-1]
    w0 = jnp.concatenate([wz_ref[0, 0, :f, :], wh_ref[0, 0, :f, :]], axis=1)
    w1 = jnp.concatenate([wz_ref[1, 0, :f, :], wh_ref[1, 0, :f, :]], axis=1)
    bcat = jnp.concatenate([bz_ref[...], bh_ref[...]], axis=1)
    g = (jnp.dot(xb, w0, precision=_DEFAULT,
                 preferred_element_type=jnp.float32)
         + jnp.dot(xb, w1, precision=_DEFAULT,
                   preferred_element_type=jnp.float32) + bcat)
    z = jax.nn.sigmoid(g[:, :hd])
    ht = jnp.tanh(g[:, hd:])
    h = (1.0 - z) * ht
    hcat = jnp.concatenate([jnp.maximum(h, 0.0), jax.nn.sigmoid(h)], axis=1)
    out = jnp.dot(hcat, wl_ref[...], precision=_DEFAULT,
                  preferred_element_type=jnp.float32) + bl_ref[...]
    o_ref[...] = jnp.maximum(out, 0.0)


def kernel(x, edge_index, edge_weight, W_z, b_z, W_r, b_r, W_h, b_h,
           W_lin, b_lin):
    del edge_index, edge_weight, W_r, b_r
    n, f = x.shape
    hdim = W_z.shape[-1]
    block = _BLOCK_ROWS if n % _BLOCK_ROWS == 0 else n
    grid = n // block

    bz2 = b_z.reshape(1, hdim)
    bh2 = b_h.reshape(1, hdim)
    bl2 = b_lin.reshape(1, 1)

    full = lambda a: pl.BlockSpec(a.shape, lambda i: (0,) * a.ndim)
    return pl.pallas_call(
        _fused_cell,
        grid=(grid,),
        in_specs=[
            pl.BlockSpec((block, f), lambda i: (i, 0)),
            full(W_z), full(bz2), full(W_h), full(bh2),
            full(W_lin), full(bl2),
        ],
        out_specs=pl.BlockSpec((block, 1), lambda i: (i, 0)),
        out_shape=jax.ShapeDtypeStruct((n, 1), jnp.float32),
        compiler_params=pltpu.CompilerParams(
            dimension_semantics=("parallel",)),
    )(x, W_z, bz2, W_h, bh2, W_lin, bl2)
